# baseline (speedup 1.0000x reference)
import functools

import jax
import jax.numpy as jnp
from jax import lax
from jax.experimental import pallas as pl
from jax.experimental.pallas import tpu as pltpu
from jax.experimental.pallas import tpu_sc as plsc

_B, _S, _D = 4, 2048, 768
_N = _B * _S
_NW = 32
_SPW = _S // _NW
_CH = 16
_NCH = (_B * _SPW) // _CH
_NBUF = 2
_LANES = _D // 16
_CPB = _SPW // _CH


def _emb_body(idx_hbm, table_hbm, pos_hbm, out_hbm,
              idx_v, pos_v, g0, g1, o0, o1,
              psem, gsem0, gsem1, ssem0, ssem1):
    gbuf = (g0, g1)
    obuf = (o0, o1)
    gsem = (gsem0, gsem1)
    ssem = (ssem0, ssem1)
    nc = 2
    wid = lax.axis_index("s") * nc + lax.axis_index("c")
    pos0 = wid * _SPW

    pcp = pltpu.async_copy(pos_hbm.at[pl.ds(pos0, _SPW)], pos_v, psem)
    for b in range(_B):
        pltpu.sync_copy(idx_hbm.at[pl.ds(b * _S + pos0, _SPW)],
                        idx_v.at[pl.ds(b * _SPW, _SPW)])

    def gather_desc(ci, slot):
        return pltpu.make_async_copy(
            table_hbm.at[idx_v.at[pl.ds(ci * _CH, _CH)]], gbuf[slot],
            gsem[slot])

    def store_desc(ci, slot):
        off = lax.div(ci, _CPB) * _S + pos0 + lax.rem(ci, _CPB) * _CH
        return pltpu.make_async_copy(
            obuf[slot], out_hbm.at[pl.ds(off, _CH)], ssem[slot])

    for slot in range(_NBUF):
        pltpu.async_copy(table_hbm.at[idx_v.at[pl.ds(slot * _CH, _CH)]],
                         gbuf[slot], gsem[slot])
    pcp.wait()

    def round_body(g, carry):
        for slot in range(_NBUF):
            ci = g * _NBUF + slot
            gather_desc(ci, slot).wait()

            @pl.when(g > 0)
            def _():
                store_desc(ci - _NBUF, slot).wait()

            p_base = lax.rem(ci, _CPB) * _CH

            def row_body(r, c2):
                for c in range(_LANES):
                    sl = pl.ds(c * 16, 16)
                    obuf[slot][r, sl] = gbuf[slot][r, sl] + pos_v[p_base + r, sl]
                return c2

            lax.fori_loop(0, _CH, row_body, 0)

            @pl.when(ci + _NBUF < _NCH)
            def _():
                pltpu.async_copy(
                    table_hbm.at[idx_v.at[pl.ds((ci + _NBUF) * _CH, _CH)]],
                    gbuf[slot], gsem[slot])

            off = lax.div(ci, _CPB) * _S + pos0 + lax.rem(ci, _CPB) * _CH
            pltpu.async_copy(obuf[slot], out_hbm.at[pl.ds(off, _CH)],
                             ssem[slot])
        return carry

    lax.fori_loop(0, _NCH // _NBUF, round_body, 0)

    for slot in range(_NBUF):
        store_desc(_NCH - _NBUF + slot, slot).wait()


@jax.jit
def _emb_lookup(idx_flat, token_table, pos_table):
    mesh = plsc.VectorSubcoreMesh(core_axis_name="c", subcore_axis_name="s")
    return pl.kernel(
        _emb_body,
        mesh=mesh,
        out_type=jax.ShapeDtypeStruct((_N, _D), jnp.float32),
        scratch_types=[
            pltpu.VMEM((_B * _SPW,), jnp.int32),
            pltpu.VMEM((_SPW, _D), jnp.float32),
            pltpu.VMEM((_CH, _D), jnp.float32),
            pltpu.VMEM((_CH, _D), jnp.float32),
            pltpu.VMEM((_CH, _D), jnp.float32),
            pltpu.VMEM((_CH, _D), jnp.float32),
            pltpu.SemaphoreType.DMA,
            pltpu.SemaphoreType.DMA,
            pltpu.SemaphoreType.DMA,
            pltpu.SemaphoreType.DMA,
            pltpu.SemaphoreType.DMA,
        ],
    )(idx_flat, token_table, pos_table)


def kernel(embedding_idx, token_table, pos_table):
    b, s = embedding_idx.shape
    idx_flat = embedding_idx.reshape(b * s).astype(jnp.int32)
    out = _emb_lookup(idx_flat, token_table, pos_table)
    return out.reshape(b, s, token_table.shape[1])


# speedup vs baseline: 1.0777x; 1.0777x over previous
import functools

import jax
import jax.numpy as jnp
from jax import lax
from jax.experimental import pallas as pl
from jax.experimental.pallas import tpu as pltpu
from jax.experimental.pallas import tpu_sc as plsc

_B, _S, _D = 4, 2048, 768
_N = _B * _S
_NW = 32
_SPW = _S // _NW
_CH = 16
_NCH = (_B * _SPW) // _CH
_NBUF = 2
_LANES = _D // 16
_CPB = _SPW // _CH


def _emb_body(idx_hbm, table_hbm, pos_hbm, out_hbm,
              idx_v, pos_v, g0, g1, o0, o1,
              psem, gsem0, gsem1, ssem0, ssem1):
    gbuf = (g0, g1)
    obuf = (o0, o1)
    gsem = (gsem0, gsem1)
    ssem = (ssem0, ssem1)
    nc = 2
    wid = lax.axis_index("s") * nc + lax.axis_index("c")
    pos0 = wid * _SPW

    pcp = pltpu.async_copy(pos_hbm.at[pl.ds(pos0, _SPW)], pos_v, psem)
    for b in range(_B):
        pltpu.sync_copy(idx_hbm.at[pl.ds(b * _S + pos0, _SPW)],
                        idx_v.at[pl.ds(b * _SPW, _SPW)])

    def issue_gather(ci, slot):
        pltpu.async_copy(table_hbm.at[idx_v.at[pl.ds(ci * _CH, _CH)]],
                         gbuf[slot], gsem[slot])

    def wait_gather(ci, slot):
        pltpu.make_async_copy(
            table_hbm.at[idx_v.at[pl.ds(ci * _CH, _CH)]], gbuf[slot],
            gsem[slot]).wait()

    def out_off(ci):
        return lax.div(ci, _CPB) * _S + pos0 + lax.rem(ci, _CPB) * _CH

    def issue_store(ci, slot):
        pltpu.async_copy(obuf[slot], out_hbm.at[pl.ds(out_off(ci), _CH)],
                         ssem[slot])

    def wait_store(ci, slot):
        pltpu.make_async_copy(
            obuf[slot], out_hbm.at[pl.ds(out_off(ci), _CH)],
            ssem[slot]).wait()

    def add_rows(ci, slot):
        p_base = lax.rem(ci, _CPB) * _CH

        def row_body(r, c2):
            for c in range(_LANES):
                sl = pl.ds(c * 16, 16)
                obuf[slot][r, sl] = gbuf[slot][r, sl] + pos_v[p_base + r, sl]
            return c2

        lax.fori_loop(0, _CH, row_body, 0)

    for slot in range(_NBUF):
        issue_gather(slot, slot)
    pcp.wait()

    for slot in range(_NBUF):
        wait_gather(slot, slot)
        add_rows(slot, slot)
        issue_gather(slot + _NBUF, slot)
        issue_store(slot, slot)

    def round_body(g, carry):
        for slot in range(_NBUF):
            ci = g * _NBUF + slot
            wait_gather(ci, slot)
            wait_store(ci - _NBUF, slot)
            add_rows(ci, slot)
            issue_gather(ci + _NBUF, slot)
            issue_store(ci, slot)
        return carry

    lax.fori_loop(1, _NCH // _NBUF - 1, round_body, 0)

    for slot in range(_NBUF):
        ci = _NCH - _NBUF + slot
        wait_gather(ci, slot)
        wait_store(ci - _NBUF, slot)
        add_rows(ci, slot)
        issue_store(ci, slot)

    for slot in range(_NBUF):
        wait_store(_NCH - _NBUF + slot, slot)


@jax.jit
def _emb_lookup(idx_flat, token_table, pos_table):
    mesh = plsc.VectorSubcoreMesh(core_axis_name="c", subcore_axis_name="s")
    return pl.kernel(
        _emb_body,
        mesh=mesh,
        out_type=jax.ShapeDtypeStruct((_N, _D), jnp.float32),
        scratch_types=[
            pltpu.VMEM((_B * _SPW,), jnp.int32),
            pltpu.VMEM((_SPW, _D), jnp.float32),
            pltpu.VMEM((_CH, _D), jnp.float32),
            pltpu.VMEM((_CH, _D), jnp.float32),
            pltpu.VMEM((_CH, _D), jnp.float32),
            pltpu.VMEM((_CH, _D), jnp.float32),
            pltpu.SemaphoreType.DMA,
            pltpu.SemaphoreType.DMA,
            pltpu.SemaphoreType.DMA,
            pltpu.SemaphoreType.DMA,
            pltpu.SemaphoreType.DMA,
        ],
    )(idx_flat, token_table, pos_table)


def kernel(embedding_idx, token_table, pos_table):
    b, s = embedding_idx.shape
    idx_flat = embedding_idx.reshape(b * s).astype(jnp.int32)
    out = _emb_lookup(idx_flat, token_table, pos_table)
    return out.reshape(b, s, token_table.shape[1])


# speedup vs baseline: 1.3728x; 1.2738x over previous
import functools

import jax
import jax.numpy as jnp
from jax import lax
from jax.experimental import pallas as pl
from jax.experimental.pallas import tpu as pltpu
from jax.experimental.pallas import tpu_sc as plsc

_B, _S, _D = 4, 2048, 768
_N = _B * _S
_NW = 32
_SPW = _S // _NW
_CH = 16
_NBUF = 4
_NCH = (_B * _SPW) // _CH
_LANES = _D // 16


def _emb_body(idx_hbm, table_hbm, pos_hbm, out_hbm,
              idx_v, pos_v, g0, g1, g2, g3,
              psem, gsem0, gsem1, gsem2, gsem3,
              ssem0, ssem1, ssem2, ssem3):
    gbuf = (g0, g1, g2, g3)
    gsem = (gsem0, gsem1, gsem2, gsem3)
    ssem = (ssem0, ssem1, ssem2, ssem3)
    nc = 2
    wid = lax.axis_index("s") * nc + lax.axis_index("c")
    pos0 = wid * _SPW

    pcp = pltpu.async_copy(pos_hbm.at[pl.ds(pos0, _SPW)], pos_v, psem)
    for b in range(_B):
        pltpu.sync_copy(idx_hbm.at[pl.ds(b * _S + pos0, _SPW)],
                        idx_v.at[pl.ds(b * _SPW, _SPW)])

    def issue_gather(ci, s):
        pltpu.async_copy(table_hbm.at[idx_v.at[pl.ds(ci * _CH, _CH)]],
                         gbuf[s], gsem[s])

    def wait_gather(ci, s):
        pltpu.make_async_copy(
            table_hbm.at[idx_v.at[pl.ds(ci * _CH, _CH)]], gbuf[s],
            gsem[s]).wait()

    def out_ref(ci, s):
        base = lax.div(ci, _NBUF) * _S + pos0 + s * _CH
        return out_hbm.at[pl.ds(base, _CH)]

    def issue_store(ci, s):
        pltpu.async_copy(gbuf[s], out_ref(ci, s), ssem[s])

    def wait_store(ci, s):
        pltpu.make_async_copy(gbuf[s], out_ref(ci, s), ssem[s]).wait()

    def add_rows(s):
        def row_body(r, c2):
            for c in range(_LANES):
                sl = pl.ds(c * 16, 16)
                plsc.addupdate(gbuf[s].at[r, sl], pos_v[s * _CH + r, sl])
            return c2

        lax.fori_loop(0, _CH, row_body, 0)

    issue_gather(0, 0)
    issue_gather(1, 1)
    pcp.wait()

    def chunk_mid(ci, s):
        wait_gather(ci, s)
        add_rows(s)
        issue_store(ci, s)
        wait_store(ci - 2, (s + 2) % _NBUF)
        issue_gather(ci + 2, (s + 2) % _NBUF)

    for s in range(2):
        wait_gather(s, s)
        add_rows(s)
        issue_store(s, s)
        issue_gather(s + 2, s + 2)
    for s in range(2, _NBUF):
        chunk_mid(s, s)

    def round_body(g, carry):
        for s in range(_NBUF):
            chunk_mid(g * _NBUF + s, s)
        return carry

    lax.fori_loop(1, _B - 1, round_body, 0)

    g0ci = (_B - 1) * _NBUF
    for s in range(2):
        chunk_mid(g0ci + s, s)
    for s in range(2, _NBUF):
        wait_gather(g0ci + s, s)
        add_rows(s)
        issue_store(g0ci + s, s)

    for s in range(_NBUF):
        wait_store(g0ci + s, s)


@jax.jit
def _emb_lookup(idx_flat, token_table, pos_table):
    mesh = plsc.VectorSubcoreMesh(core_axis_name="c", subcore_axis_name="s")
    return pl.kernel(
        _emb_body,
        mesh=mesh,
        out_type=jax.ShapeDtypeStruct((_N, _D), jnp.float32),
        scratch_types=[
            pltpu.VMEM((_B * _SPW,), jnp.int32),
            pltpu.VMEM((_SPW, _D), jnp.float32),
            pltpu.VMEM((_CH, _D), jnp.float32),
            pltpu.VMEM((_CH, _D), jnp.float32),
            pltpu.VMEM((_CH, _D), jnp.float32),
            pltpu.VMEM((_CH, _D), jnp.float32),
            pltpu.SemaphoreType.DMA,
            pltpu.SemaphoreType.DMA,
            pltpu.SemaphoreType.DMA,
            pltpu.SemaphoreType.DMA,
            pltpu.SemaphoreType.DMA,
            pltpu.SemaphoreType.DMA,
            pltpu.SemaphoreType.DMA,
            pltpu.SemaphoreType.DMA,
            pltpu.SemaphoreType.DMA,
        ],
    )(idx_flat, token_table, pos_table)


def kernel(embedding_idx, token_table, pos_table):
    b, s = embedding_idx.shape
    idx_flat = embedding_idx.reshape(b * s).astype(jnp.int32)
    out = _emb_lookup(idx_flat, token_table, pos_table)
    return out.reshape(b, s, token_table.shape[1])


# speedup vs baseline: 1.5705x; 1.1440x over previous
import functools

import jax
import jax.numpy as jnp
from jax import lax
from jax.experimental import pallas as pl
from jax.experimental.pallas import tpu as pltpu
from jax.experimental.pallas import tpu_sc as plsc

_B, _S, _D = 4, 2048, 768
_N = _B * _S
_NW = 32
_SPW = _S // _NW
_CH = 16
_NBUF = 4
_NCH = (_B * _SPW) // _CH
_LANES = _D // 16


def _emb_body(idx_hbm, table_hbm, pos_hbm, out_hbm,
              idx_v, pos_v, g0, g1, g2, g3,
              psem, gsem0, gsem1, gsem2, gsem3,
              ssem0, ssem1, ssem2, ssem3):
    gbuf = (g0, g1, g2, g3)
    gsem = (gsem0, gsem1, gsem2, gsem3)
    ssem = (ssem0, ssem1, ssem2, ssem3)
    nc = 2
    wid = lax.axis_index("s") * nc + lax.axis_index("c")
    pos0 = wid * _SPW

    pcp = pltpu.async_copy(pos_hbm.at[pl.ds(pos0, _SPW)], pos_v, psem)
    for b in range(_B):
        pltpu.sync_copy(idx_hbm.at[pl.ds(b * _S + pos0, _SPW)],
                        idx_v.at[pl.ds(b * _SPW, _SPW)])

    def issue_gather(ci, s):
        pltpu.async_copy(table_hbm.at[idx_v.at[pl.ds(ci * _CH, _CH)]],
                         gbuf[s], gsem[s])

    def wait_gather(ci, s):
        pltpu.make_async_copy(
            table_hbm.at[idx_v.at[pl.ds(ci * _CH, _CH)]], gbuf[s],
            gsem[s]).wait()

    def out_ref(ci, s):
        base = lax.div(ci, _NBUF) * _S + pos0 + s * _CH
        return out_hbm.at[pl.ds(base, _CH)]

    def issue_store(ci, s):
        pltpu.async_copy(gbuf[s], out_ref(ci, s), ssem[s])

    def wait_store(ci, s):
        pltpu.make_async_copy(gbuf[s], out_ref(ci, s), ssem[s]).wait()

    def add_rows(s):
        @plsc.parallel_loop(0, _CH)
        def row_body(r):
            for c in range(_LANES):
                sl = pl.ds(c * 16, 16)
                plsc.addupdate(gbuf[s].at[r, sl], pos_v[s * _CH + r, sl])

    issue_gather(0, 0)
    issue_gather(1, 1)
    pcp.wait()

    def chunk_mid(ci, s):
        wait_gather(ci, s)
        add_rows(s)
        issue_store(ci, s)
        wait_store(ci - 2, (s + 2) % _NBUF)
        issue_gather(ci + 2, (s + 2) % _NBUF)

    for s in range(2):
        wait_gather(s, s)
        add_rows(s)
        issue_store(s, s)
        issue_gather(s + 2, s + 2)
    for s in range(2, _NBUF):
        chunk_mid(s, s)

    def round_body(g, carry):
        for s in range(_NBUF):
            chunk_mid(g * _NBUF + s, s)
        return carry

    lax.fori_loop(1, _B - 1, round_body, 0)

    g0ci = (_B - 1) * _NBUF
    for s in range(2):
        chunk_mid(g0ci + s, s)
    for s in range(2, _NBUF):
        wait_gather(g0ci + s, s)
        add_rows(s)
        issue_store(g0ci + s, s)

    for s in range(_NBUF):
        wait_store(g0ci + s, s)


@jax.jit
def _emb_lookup(idx_flat, token_table, pos_table):
    mesh = plsc.VectorSubcoreMesh(core_axis_name="c", subcore_axis_name="s")
    return pl.kernel(
        _emb_body,
        mesh=mesh,
        out_type=jax.ShapeDtypeStruct((_N, _D), jnp.float32),
        scratch_types=[
            pltpu.VMEM((_B * _SPW,), jnp.int32),
            pltpu.VMEM((_SPW, _D), jnp.float32),
            pltpu.VMEM((_CH, _D), jnp.float32),
            pltpu.VMEM((_CH, _D), jnp.float32),
            pltpu.VMEM((_CH, _D), jnp.float32),
            pltpu.VMEM((_CH, _D), jnp.float32),
            pltpu.SemaphoreType.DMA,
            pltpu.SemaphoreType.DMA,
            pltpu.SemaphoreType.DMA,
            pltpu.SemaphoreType.DMA,
            pltpu.SemaphoreType.DMA,
            pltpu.SemaphoreType.DMA,
            pltpu.SemaphoreType.DMA,
            pltpu.SemaphoreType.DMA,
            pltpu.SemaphoreType.DMA,
        ],
    )(idx_flat, token_table, pos_table)


def kernel(embedding_idx, token_table, pos_table):
    b, s = embedding_idx.shape
    idx_flat = embedding_idx.reshape(b * s).astype(jnp.int32)
    out = _emb_lookup(idx_flat, token_table, pos_table)
    return out.reshape(b, s, token_table.shape[1])


# speedup vs baseline: 1.5925x; 1.0140x over previous
import functools

import jax
import jax.numpy as jnp
from jax import lax
from jax.experimental import pallas as pl
from jax.experimental.pallas import tpu as pltpu
from jax.experimental.pallas import tpu_sc as plsc

_B, _S, _D = 4, 2048, 768
_N = _B * _S
_NW = 32
_SPW = _S // _NW
_WR = 8
_NWIN = _SPW // _WR
_NQ = 3
_LANES = _D // 16


def _emb_body(idx_hbm, table_hbm, pos_hbm, out_hbm,
              idx_v, pos_v, *bufs_and_sems):
    qbuf = [[bufs_and_sems[s * _B + b] for b in range(_B)]
            for s in range(_NQ)]
    base = _NQ * _B
    gsem = [[bufs_and_sems[base + s * _B + b] for b in range(_B)]
            for s in range(_NQ)]
    base += _NQ * _B
    ssem = [[bufs_and_sems[base + s * _B + b] for b in range(_B)]
            for s in range(_NQ)]
    psem = bufs_and_sems[base + _NQ * _B]
    nc = 2
    wid = lax.axis_index("s") * nc + lax.axis_index("c")
    pos0 = wid * _SPW

    pcp = pltpu.async_copy(pos_hbm.at[pl.ds(pos0, _SPW)], pos_v, psem)
    for b in range(_B):
        pltpu.sync_copy(idx_hbm.at[pl.ds(b * _S + pos0, _SPW)],
                        idx_v.at[pl.ds(b * _SPW, _SPW)])

    def idx_sl(w, b):
        return idx_v.at[pl.ds(b * _SPW + w * _WR, _WR)]

    def issue_gathers(w, s):
        for b in range(_B):
            pltpu.async_copy(table_hbm.at[idx_sl(w, b)], qbuf[s][b],
                             gsem[s][b])

    def wait_gathers(w, s):
        for b in range(_B):
            pltpu.make_async_copy(table_hbm.at[idx_sl(w, b)], qbuf[s][b],
                                  gsem[s][b]).wait()

    def out_ref(w, b):
        return out_hbm.at[pl.ds(b * _S + pos0 + w * _WR, _WR)]

    def issue_stores(w, s):
        for b in range(_B):
            pltpu.async_copy(qbuf[s][b], out_ref(w, b), ssem[s][b])

    def wait_stores(w, s):
        for b in range(_B):
            pltpu.make_async_copy(qbuf[s][b], out_ref(w, b),
                                  ssem[s][b]).wait()

    def add_rows(w, s):
        @plsc.parallel_loop(0, _WR)
        def row_body(r):
            for c in range(_LANES):
                sl = pl.ds(c * 16, 16)
                v = pos_v[w * _WR + r, sl]
                for b in range(_B):
                    plsc.addupdate(qbuf[s][b].at[r, sl], v)

    issue_gathers(0, 0)
    issue_gathers(1, 1)
    pcp.wait()

    for w in range(_NWIN):
        s = w % _NQ
        wait_gathers(w, s)
        add_rows(w, s)
        issue_stores(w, s)
        if w + 2 < _NWIN:
            if w >= 1:
                wait_stores(w - 1, (w + 2) % _NQ)
            issue_gathers(w + 2, (w + 2) % _NQ)

    for w in range(_NWIN - _NQ, _NWIN):
        wait_stores(w, w % _NQ)


@jax.jit
def _emb_lookup(idx_flat, token_table, pos_table):
    mesh = plsc.VectorSubcoreMesh(core_axis_name="c", subcore_axis_name="s")
    scratch = [
        pltpu.VMEM((_B * _SPW,), jnp.int32),
        pltpu.VMEM((_SPW, _D), jnp.float32),
    ]
    scratch += [pltpu.VMEM((_WR, _D), jnp.float32)
                for _ in range(_NQ * _B)]
    scratch += [pltpu.SemaphoreType.DMA
                for _ in range(2 * _NQ * _B)]
    scratch += [pltpu.SemaphoreType.DMA]
    return pl.kernel(
        _emb_body,
        mesh=mesh,
        out_type=jax.ShapeDtypeStruct((_N, _D), jnp.float32),
        scratch_types=scratch,
    )(idx_flat, token_table, pos_table)


def kernel(embedding_idx, token_table, pos_table):
    b, s = embedding_idx.shape
    idx_flat = embedding_idx.reshape(b * s).astype(jnp.int32)
    out = _emb_lookup(idx_flat, token_table, pos_table)
    return out.reshape(b, s, token_table.shape[1])
